# Initial kernel scaffold; baseline (speedup 1.0000x reference)
import jax
import jax.numpy as jnp
from jax.experimental import pallas as pl


def kernel(sequence, table):
    raise NotImplementedError("write your pallas kernel here")


# speedup vs baseline: 3.4129x; 3.4129x over previous
import jax
import jax.numpy as jnp
from jax.experimental import pallas as pl


def _copy_body(tab_ref, out_ref):
    out_ref[...] = tab_ref[...][None, :, :]


def kernel(sequence, table):
    batch, seq_len = sequence.shape
    hidden = table.shape[1]
    rows = 512
    grid = (seq_len // rows, batch)

    out = pl.pallas_call(
        _copy_body,
        grid=grid,
        in_specs=[pl.BlockSpec((rows, hidden), lambda i, b: (i, 0))],
        out_specs=pl.BlockSpec((1, rows, hidden), lambda i, b: (b, i, 0)),
        out_shape=jax.ShapeDtypeStruct((batch, seq_len, hidden), table.dtype),
    )(table)
    return out


# speedup vs baseline: 3.5248x; 1.0328x over previous
import functools

import jax
import jax.numpy as jnp
from jax import lax
from jax.experimental import pallas as pl
from jax.experimental.pallas import tpu as pltpu
from jax.experimental.pallas import tpu_sc as plsc

_NUM_CORES = 2
_NUM_SUBCORES = 16
_NUM_WORKERS = _NUM_CORES * _NUM_SUBCORES
_CHUNK = 32


def _sc_body(batch, rows_per_worker, table_hbm, out_hbm,
             buf0, buf1, lsem0, lsem1, ssem0, ssem1):
    wid = lax.axis_index("s") * _NUM_CORES + lax.axis_index("c")
    base = wid * rows_per_worker
    bufs = (buf0, buf1)
    lsems = (lsem0, lsem1)
    ssems = (ssem0, ssem1)
    nch = rows_per_worker // _CHUNK

    loads = [None] * nch
    stores = [[] for _ in range(nch)]
    loads[0] = pltpu.async_copy(table_hbm.at[pl.ds(base, _CHUNK)], buf0, lsem0)
    for c in range(nch):
        pb = c % 2
        if c + 1 < nch:
            for d in stores[c - 1] if c >= 1 else ():
                d.wait()
            loads[c + 1] = pltpu.async_copy(
                table_hbm.at[pl.ds(base + (c + 1) * _CHUNK, _CHUNK)],
                bufs[(c + 1) % 2], lsems[(c + 1) % 2])
        loads[c].wait()
        r0 = base + c * _CHUNK
        for b in range(batch):
            stores[c].append(pltpu.async_copy(
                bufs[pb], out_hbm.at[b, pl.ds(r0, _CHUNK)], ssems[pb]))
    for c in (nch - 2, nch - 1):
        if c >= 0:
            for d in stores[c]:
                d.wait()


def kernel(sequence, table):
    batch, seq_len = sequence.shape
    hidden = table.shape[1]
    rows_per_worker = seq_len // _NUM_WORKERS

    mesh = plsc.VectorSubcoreMesh(core_axis_name="c", subcore_axis_name="s")
    sc_kernel = pl.kernel(
        functools.partial(_sc_body, batch, rows_per_worker),
        out_type=jax.ShapeDtypeStruct((batch, seq_len, hidden), table.dtype),
        mesh=mesh,
        scratch_types=[
            pltpu.VMEM((_CHUNK, hidden), table.dtype),
            pltpu.VMEM((_CHUNK, hidden), table.dtype),
            pltpu.SemaphoreType.DMA,
            pltpu.SemaphoreType.DMA,
            pltpu.SemaphoreType.DMA,
            pltpu.SemaphoreType.DMA,
        ],
    )
    return sc_kernel(table)
